# Initial kernel scaffold; baseline (speedup 1.0000x reference)
import jax
import jax.numpy as jnp
from jax.experimental import pallas as pl


def kernel(inputs, embedding):
    raise NotImplementedError("write your pallas kernel here")


# speedup vs baseline: 4.0824x; 4.0824x over previous
import functools

import jax
import jax.numpy as jnp
from jax import lax
from jax.experimental import pallas as pl
from jax.experimental.pallas import tpu as pltpu
from jax.experimental.pallas import tpu_sc as plsc

_NC = 2
_NS = 16
_NW = _NC * _NS

_CH = 128


def _embed_gather(table, idx_flat):
    (B,) = idx_flat.shape
    V, D = table.shape
    assert B % _NW == 0
    b_per_w = B // _NW
    assert b_per_w % _CH == 0
    n_ch = b_per_w // _CH
    mesh = plsc.VectorSubcoreMesh(core_axis_name="c", subcore_axis_name="s")

    @functools.partial(
        pl.kernel,
        mesh=mesh,
        out_type=jax.ShapeDtypeStruct((B, D), jnp.float32),
        scratch_types=[
            pltpu.VMEM((b_per_w,), jnp.int32),
            pltpu.VMEM((_CH, D), jnp.float32),
            pltpu.SemaphoreType.DMA,
        ],
        compiler_params=pltpu.CompilerParams(use_tc_tiling_on_sc=False),
    )
    def k(table_hbm, idx_hbm, out_hbm, idx_v, rows_v, sem):
        wid = lax.axis_index("s") * _NC + lax.axis_index("c")
        base = wid * b_per_w
        pltpu.sync_copy(idx_hbm.at[pl.ds(base, b_per_w)], idx_v)

        def body(j, carry):
            off = j * _CH
            pltpu.async_copy(
                table_hbm.at[idx_v.at[pl.ds(off, _CH)]], rows_v, sem
            ).wait()
            pltpu.sync_copy(rows_v, out_hbm.at[pl.ds(base + off, _CH)])
            return carry

        lax.fori_loop(0, n_ch, body, 0)

    return k(table, idx_flat)


def kernel(inputs, embedding):
    B0, B1 = inputs.shape
    idx_flat = inputs.reshape(B0 * B1).astype(jnp.int32)
    out = _embed_gather(embedding, idx_flat)
    return out.reshape(B0, B1, embedding.shape[1])


# speedup vs baseline: 4.6754x; 1.1453x over previous
import functools

import jax
import jax.numpy as jnp
from jax import lax
from jax.experimental import pallas as pl
from jax.experimental.pallas import tpu as pltpu
from jax.experimental.pallas import tpu_sc as plsc

_NC = 2
_NS = 16
_NW = _NC * _NS

_CH = 128


_NBUF = 10


def _embed_gather(table, idx_flat):
    (B,) = idx_flat.shape
    V, D = table.shape
    assert B % _NW == 0
    b_per_w = B // _NW
    assert b_per_w % _CH == 0
    n_ch = b_per_w // _CH
    assert n_ch % _NBUF == 0 and n_ch >= 2 * _NBUF
    n_grp = n_ch // _NBUF
    mesh = plsc.VectorSubcoreMesh(core_axis_name="c", subcore_axis_name="s")

    @functools.partial(
        pl.kernel,
        mesh=mesh,
        out_type=jax.ShapeDtypeStruct((B, D), jnp.float32),
        scratch_types=[
            pltpu.VMEM((b_per_w,), jnp.int32),
            pltpu.VMEM((_NBUF, _CH, D), jnp.float32),
            pltpu.SemaphoreType.DMA((_NBUF,)),
            pltpu.SemaphoreType.DMA((_NBUF,)),
        ],
        compiler_params=pltpu.CompilerParams(use_tc_tiling_on_sc=False),
    )
    def k(table_hbm, idx_hbm, out_hbm, idx_v, rows_v, gsem, osem):
        wid = lax.axis_index("s") * _NC + lax.axis_index("c")
        base = wid * b_per_w
        pltpu.sync_copy(idx_hbm.at[pl.ds(base, b_per_w)], idx_v)

        def gather(j, b):
            pltpu.async_copy(
                table_hbm.at[idx_v.at[pl.ds(j * _CH, _CH)]],
                rows_v.at[b],
                gsem.at[b],
            )

        def write(j, b):
            pltpu.async_copy(
                rows_v.at[b], out_hbm.at[pl.ds(base + j * _CH, _CH)], osem.at[b]
            )

        for b in range(_NBUF):
            gather(b, b)

        def group(gi, carry):
            g = gi * _NBUF
            for b in range(_NBUF):
                pltpu.make_async_copy(
                    table_hbm.at[idx_v.at[pl.ds(0, _CH)]], rows_v.at[b], gsem.at[b]
                ).wait()
                write(g + b, b)
            for b in range(_NBUF):
                pltpu.make_async_copy(
                    rows_v.at[b], out_hbm.at[pl.ds(base, _CH)], osem.at[b]
                ).wait()
                gather(g + _NBUF + b, b)
            return carry

        lax.fori_loop(0, n_grp - 1, group, 0)

        g = (n_grp - 1) * _NBUF
        for b in range(_NBUF):
            pltpu.make_async_copy(
                table_hbm.at[idx_v.at[pl.ds(0, _CH)]], rows_v.at[b], gsem.at[b]
            ).wait()
            write(g + b, b)
        for b in range(_NBUF):
            pltpu.make_async_copy(
                rows_v.at[b], out_hbm.at[pl.ds(base, _CH)], osem.at[b]
            ).wait()

    return k(table, idx_flat)


def kernel(inputs, embedding):
    B0, B1 = inputs.shape
    idx_flat = inputs.reshape(B0 * B1).astype(jnp.int32)
    out = _embed_gather(embedding, idx_flat)
    return out.reshape(B0, B1, embedding.shape[1])
